# Initial kernel scaffold; baseline (speedup 1.0000x reference)
import functools

import jax
import jax.numpy as jnp
from jax import lax
from jax.experimental import pallas as pl
from jax.experimental.pallas import tpu as pltpu
from jax.experimental.pallas import tpu_sc as plsc

_N = 10000
_E = 320000
_H = 128
_G = 64
_C = 10

_NC = 2
_NS = 16
_NW = _NC * _NS
_EPT = _E // _NW
_EBATCH = 125
_NCH = _EPT // _EBATCH
_ZR = _N // _NS

_mesh = plsc.VectorSubcoreMesh(
    core_axis_name="c", subcore_axis_name="s", num_cores=_NC, num_subcores=_NS
)


@functools.partial(
    pl.kernel,
    out_type=jax.ShapeDtypeStruct((_NC, _N, _H), jnp.float32),
    mesh=_mesh,
    scratch_types=[
        pltpu.VMEM((_NCH, _EBATCH), jnp.int32),
        pltpu.VMEM((_NCH, _EBATCH), jnp.int32),
        pltpu.VMEM((_EBATCH, _H), jnp.float32),
        pltpu.VMEM_SHARED((_N, _H), jnp.float32),
        pltpu.SemaphoreType.DMA,
    ],
)
def _seg_sum(h, src, dst, zeros, out, src_v, dst_v, rows_v, acc, sem):
    c = lax.axis_index("c")
    s = lax.axis_index("s")
    wid = c * _NS + s
    pltpu.sync_copy(zeros, acc.at[pl.ds(s * _ZR, _ZR)])
    pltpu.sync_copy(src.at[pl.ds(wid * _NCH, _NCH)], src_v)
    pltpu.sync_copy(dst.at[pl.ds(wid * _NCH, _NCH)], dst_v)
    plsc.subcore_barrier()

    def body(j, carry):
        pltpu.async_copy(h.at[src_v.at[j]], rows_v, sem).wait()
        pltpu.sync_copy(rows_v, acc.at[dst_v.at[j]], add=True)
        return carry

    lax.fori_loop(0, _NCH, body, 0)
    plsc.subcore_barrier()
    pltpu.sync_copy(acc.at[pl.ds(s * _ZR, _ZR)], out.at[c].at[pl.ds(s * _ZR, _ZR)])


_BR = 500


def _lin_body(a0, a1, x, wr, ws, b, o, *, relu):
    y = jnp.dot(a0[...] + a1[...], wr[...], preferred_element_type=jnp.float32)
    y = y + jnp.dot(x[...], ws[...], preferred_element_type=jnp.float32)
    y = y + b[...]
    if relu:
        y = jnp.maximum(y, 0.0)
    o[...] = y


def _fused_linear(a0, a1, x, wr, ws, b, relu):
    return pl.pallas_call(
        functools.partial(_lin_body, relu=relu),
        grid=(_N // _BR,),
        in_specs=[
            pl.BlockSpec((_BR, _H), lambda i: (i, 0)),
            pl.BlockSpec((_BR, _H), lambda i: (i, 0)),
            pl.BlockSpec((_BR, _H), lambda i: (i, 0)),
            pl.BlockSpec((_H, _H), lambda i: (0, 0)),
            pl.BlockSpec((_H, _H), lambda i: (0, 0)),
            pl.BlockSpec((1, _H), lambda i: (0, 0)),
        ],
        out_specs=pl.BlockSpec((_BR, _H), lambda i: (i, 0)),
        out_shape=jax.ShapeDtypeStruct((_N, _H), jnp.float32),
    )(a0, a1, x, wr, ws, b)


def _pool_body(h, batch, wlin, blin, o, sums, counts):
    i = pl.program_id(0)

    @pl.when(i == 0)
    def _():
        sums[...] = jnp.zeros_like(sums)
        counts[...] = jnp.zeros_like(counts)

    oh = (lax.broadcasted_iota(jnp.int32, (_BR, _G), 1) == batch[...]).astype(
        jnp.float32
    )
    sums[...] += lax.dot_general(
        oh, h[...], (((0,), (0,)), ((), ())), preferred_element_type=jnp.float32
    )
    counts[...] += lax.dot_general(
        oh,
        jnp.ones((_BR, 1), jnp.float32),
        (((0,), (0,)), ((), ())),
        preferred_element_type=jnp.float32,
    )

    @pl.when(i == pl.num_programs(0) - 1)
    def _():
        pooled = sums[...] / jnp.maximum(counts[...], 1.0)
        o[...] = (
            jnp.dot(pooled, wlin[...], preferred_element_type=jnp.float32) + blin[...]
        )


def _pool(h, batch2d, wlin, blin):
    return pl.pallas_call(
        _pool_body,
        grid=(_N // _BR,),
        in_specs=[
            pl.BlockSpec((_BR, _H), lambda i: (i, 0)),
            pl.BlockSpec((_BR, 1), lambda i: (i, 0)),
            pl.BlockSpec((_H, _C), lambda i: (0, 0)),
            pl.BlockSpec((1, _C), lambda i: (0, 0)),
        ],
        out_specs=pl.BlockSpec((_G, _C), lambda i: (0, 0)),
        out_shape=jax.ShapeDtypeStruct((_G, _C), jnp.float32),
        scratch_shapes=[
            pltpu.VMEM((_G, _H), jnp.float32),
            pltpu.VMEM((_G, 1), jnp.float32),
        ],
    )(h, batch2d, wlin, blin)


def kernel(x, edge_index, batch, W1r, W1s, b1, W2r, W2s, b2, W3r, W3s, b3, Wlin, blin):
    src2d = edge_index[0].reshape(_E // _EBATCH, _EBATCH)
    dst2d = edge_index[1].reshape(_E // _EBATCH, _EBATCH)
    zeros = jnp.zeros((_ZR, _H), jnp.float32)
    batch2d = batch.reshape(_N, 1).astype(jnp.int32)

    h = x
    for wr, ws, b, relu in (
        (W1r, W1s, b1, True),
        (W2r, W2s, b2, True),
        (W3r, W3s, b3, False),
    ):
        accs = _seg_sum(h, src2d, dst2d, zeros)
        h = _fused_linear(accs[0], accs[1], h, wr, ws, b.reshape(1, _H), relu)
    return _pool(h, batch2d, Wlin, blin.reshape(1, _C))


# speedup vs baseline: 7.3052x; 7.3052x over previous
import functools

import jax
import jax.numpy as jnp
from jax import lax
from jax.experimental import pallas as pl
from jax.experimental.pallas import tpu as pltpu
from jax.experimental.pallas import tpu_sc as plsc

_N = 10000
_E = 320000
_H = 128
_G = 64
_C = 10

_NC = 2
_NS = 16
_NW = _NC * _NS
_EPT = _E // _NW
_EBATCH = 125
_NCH = _EPT // _EBATCH
_ZR = _N // _NS

_mesh = plsc.VectorSubcoreMesh(
    core_axis_name="c", subcore_axis_name="s", num_cores=_NC, num_subcores=_NS
)


@functools.partial(
    pl.kernel,
    out_type=jax.ShapeDtypeStruct((_NC, _N, _H), jnp.float32),
    mesh=_mesh,
    scratch_types=[
        pltpu.VMEM((_NCH, _EBATCH), jnp.int32),
        pltpu.VMEM((_NCH, _EBATCH), jnp.int32),
        pltpu.VMEM((_EBATCH, _H), jnp.float32),
        pltpu.VMEM_SHARED((_N, _H), jnp.float32),
        pltpu.SemaphoreType.DMA,
    ],
)
def _seg_sum(h, src, dst, zeros, out, src_v, dst_v, rows_v, acc, sem):
    c = lax.axis_index("c")
    s = lax.axis_index("s")
    wid = c * _NS + s
    pltpu.sync_copy(zeros, acc.at[pl.ds(s * _ZR, _ZR)])
    pltpu.sync_copy(src.at[pl.ds(wid * _NCH, _NCH)], src_v)
    pltpu.sync_copy(dst.at[pl.ds(wid * _NCH, _NCH)], dst_v)
    plsc.subcore_barrier()

    def body(j, carry):
        pltpu.async_copy(h.at[src_v.at[j]], rows_v, sem).wait()
        pltpu.sync_copy(rows_v, acc.at[dst_v.at[j]], add=True)
        return carry

    lax.fori_loop(0, _NCH, body, 0)
    plsc.subcore_barrier()

    @pl.when(s == 0)
    def _():
        pltpu.sync_copy(acc, out.at[c])


_BR = 1000


def _lin_body(a0, a1, x, wr, ws, b, o, *, relu):
    y = jnp.dot(a0[...] + a1[...], wr[...], preferred_element_type=jnp.float32)
    y = y + jnp.dot(x[...], ws[...], preferred_element_type=jnp.float32)
    y = y + b[...]
    if relu:
        y = jnp.maximum(y, 0.0)
    o[...] = y


def _fused_linear(a0, a1, x, wr, ws, b, relu):
    return pl.pallas_call(
        functools.partial(_lin_body, relu=relu),
        grid=(_N // _BR,),
        in_specs=[
            pl.BlockSpec((_BR, _H), lambda i: (i, 0)),
            pl.BlockSpec((_BR, _H), lambda i: (i, 0)),
            pl.BlockSpec((_BR, _H), lambda i: (i, 0)),
            pl.BlockSpec((_H, _H), lambda i: (0, 0)),
            pl.BlockSpec((_H, _H), lambda i: (0, 0)),
            pl.BlockSpec((1, _H), lambda i: (0, 0)),
        ],
        out_specs=pl.BlockSpec((_BR, _H), lambda i: (i, 0)),
        out_shape=jax.ShapeDtypeStruct((_N, _H), jnp.float32),
    )(a0, a1, x, wr, ws, b)


def _pool_body(h, batch, wlin, blin, o, sums, counts):
    i = pl.program_id(0)

    @pl.when(i == 0)
    def _():
        sums[...] = jnp.zeros_like(sums)
        counts[...] = jnp.zeros_like(counts)

    oh = (lax.broadcasted_iota(jnp.int32, (_BR, _G), 1) == batch[...]).astype(
        jnp.float32
    )
    sums[...] += lax.dot_general(
        oh, h[...], (((0,), (0,)), ((), ())), preferred_element_type=jnp.float32
    )
    counts[...] += lax.dot_general(
        oh,
        jnp.ones((_BR, 1), jnp.float32),
        (((0,), (0,)), ((), ())),
        preferred_element_type=jnp.float32,
    )

    @pl.when(i == pl.num_programs(0) - 1)
    def _():
        pooled = sums[...] / jnp.maximum(counts[...], 1.0)
        o[...] = (
            jnp.dot(pooled, wlin[...], preferred_element_type=jnp.float32) + blin[...]
        )


def _pool(h, batch2d, wlin, blin):
    return pl.pallas_call(
        _pool_body,
        grid=(_N // _BR,),
        in_specs=[
            pl.BlockSpec((_BR, _H), lambda i: (i, 0)),
            pl.BlockSpec((_BR, 1), lambda i: (i, 0)),
            pl.BlockSpec((_H, _C), lambda i: (0, 0)),
            pl.BlockSpec((1, _C), lambda i: (0, 0)),
        ],
        out_specs=pl.BlockSpec((_G, _C), lambda i: (0, 0)),
        out_shape=jax.ShapeDtypeStruct((_G, _C), jnp.float32),
        scratch_shapes=[
            pltpu.VMEM((_G, _H), jnp.float32),
            pltpu.VMEM((_G, 1), jnp.float32),
        ],
    )(h, batch2d, wlin, blin)


def kernel(x, edge_index, batch, W1r, W1s, b1, W2r, W2s, b2, W3r, W3s, b3, Wlin, blin):
    src2d = edge_index[0].reshape(_E // _EBATCH, _EBATCH)
    dst2d = edge_index[1].reshape(_E // _EBATCH, _EBATCH)
    zeros = jnp.zeros((_ZR, _H), jnp.float32)
    batch2d = batch.reshape(_N, 1).astype(jnp.int32)

    h = x
    for wr, ws, b, relu in (
        (W1r, W1s, b1, True),
        (W2r, W2s, b2, True),
        (W3r, W3s, b3, False),
    ):
        accs = _seg_sum(h, src2d, dst2d, zeros)
        h = _fused_linear(accs[0], accs[1], h, wr, ws, b.reshape(1, _H), relu)
    return _pool(h, batch2d, Wlin, blin.reshape(1, _C))


# speedup vs baseline: 10.6326x; 1.4555x over previous
import functools

import jax
import jax.numpy as jnp
from jax import lax
from jax.experimental import pallas as pl
from jax.experimental.pallas import tpu as pltpu
from jax.experimental.pallas import tpu_sc as plsc

_N = 10000
_E = 320000
_H = 128
_G = 64
_C = 10

_NC = 2
_NS = 16
_NW = _NC * _NS
_EPT = _E // _NW
_EBATCH = 125
_NCH = _EPT // _EBATCH
_HCH = _NCH // 2
_ZR = _N // _NS

_mesh = plsc.VectorSubcoreMesh(
    core_axis_name="c", subcore_axis_name="s", num_cores=_NC, num_subcores=_NS
)


@functools.partial(
    pl.kernel,
    out_type=jax.ShapeDtypeStruct((_NC, _N, _H), jnp.float32),
    mesh=_mesh,
    scratch_types=[
        pltpu.VMEM((2 * _HCH, _EBATCH), jnp.int32),
        pltpu.VMEM((2, _EBATCH, _H), jnp.float32),
        pltpu.VMEM_SHARED((_N, _H), jnp.float32),
        pltpu.SemaphoreType.DMA((2,)),
    ],
)
def _seg_sum(h, sd, zeros, out, idx_v, rows_v, acc, sem):
    c = lax.axis_index("c")
    s = lax.axis_index("s")
    wid = c * _NS + s
    pltpu.sync_copy(zeros, acc.at[pl.ds(s * _ZR, _ZR)])
    plsc.subcore_barrier()

    for half in range(2):
        pltpu.sync_copy(sd.at[half].at[wid], idx_v)
        pltpu.async_copy(h.at[idx_v.at[0]], rows_v.at[0], sem.at[0])

        def body(j, carry):
            nxt = j + 1

            @pl.when(nxt < _HCH)
            def _():
                nb = lax.rem(nxt, 2)
                pltpu.async_copy(h.at[idx_v.at[nxt]], rows_v.at[nb], sem.at[nb])

            jb = lax.rem(j, 2)
            pltpu.make_async_copy(
                h.at[idx_v.at[j]], rows_v.at[jb], sem.at[jb]
            ).wait()
            pltpu.sync_copy(rows_v.at[jb], acc.at[idx_v.at[_HCH + j]], add=True)
            return carry

        lax.fori_loop(0, _HCH, body, 0)
    plsc.subcore_barrier()

    @pl.when(s == 0)
    def _():
        pltpu.sync_copy(acc, out.at[c])


_BR = 1000


def _lin_body(a0, a1, x, wr, ws, b, o, *, relu):
    y = jnp.dot(a0[...] + a1[...], wr[...], preferred_element_type=jnp.float32)
    y = y + jnp.dot(x[...], ws[...], preferred_element_type=jnp.float32)
    y = y + b[...]
    if relu:
        y = jnp.maximum(y, 0.0)
    o[...] = y


def _fused_linear(a0, a1, x, wr, ws, b, relu):
    return pl.pallas_call(
        functools.partial(_lin_body, relu=relu),
        grid=(_N // _BR,),
        in_specs=[
            pl.BlockSpec((_BR, _H), lambda i: (i, 0)),
            pl.BlockSpec((_BR, _H), lambda i: (i, 0)),
            pl.BlockSpec((_BR, _H), lambda i: (i, 0)),
            pl.BlockSpec((_H, _H), lambda i: (0, 0)),
            pl.BlockSpec((_H, _H), lambda i: (0, 0)),
            pl.BlockSpec((1, _H), lambda i: (0, 0)),
        ],
        out_specs=pl.BlockSpec((_BR, _H), lambda i: (i, 0)),
        out_shape=jax.ShapeDtypeStruct((_N, _H), jnp.float32),
    )(a0, a1, x, wr, ws, b)


def _pool_body(h, batch, wlin, blin, o, sums, counts):
    i = pl.program_id(0)

    @pl.when(i == 0)
    def _():
        sums[...] = jnp.zeros_like(sums)
        counts[...] = jnp.zeros_like(counts)

    oh = (lax.broadcasted_iota(jnp.int32, (_BR, _G), 1) == batch[...]).astype(
        jnp.float32
    )
    sums[...] += lax.dot_general(
        oh, h[...], (((0,), (0,)), ((), ())), preferred_element_type=jnp.float32
    )
    counts[...] += lax.dot_general(
        oh,
        jnp.ones((_BR, 1), jnp.float32),
        (((0,), (0,)), ((), ())),
        preferred_element_type=jnp.float32,
    )

    @pl.when(i == pl.num_programs(0) - 1)
    def _():
        pooled = sums[...] / jnp.maximum(counts[...], 1.0)
        o[...] = (
            jnp.dot(pooled, wlin[...], preferred_element_type=jnp.float32) + blin[...]
        )


def _pool(h, batch2d, wlin, blin):
    return pl.pallas_call(
        _pool_body,
        grid=(_N // _BR,),
        in_specs=[
            pl.BlockSpec((_BR, _H), lambda i: (i, 0)),
            pl.BlockSpec((_BR, 1), lambda i: (i, 0)),
            pl.BlockSpec((_H, _C), lambda i: (0, 0)),
            pl.BlockSpec((1, _C), lambda i: (0, 0)),
        ],
        out_specs=pl.BlockSpec((_G, _C), lambda i: (0, 0)),
        out_shape=jax.ShapeDtypeStruct((_G, _C), jnp.float32),
        scratch_shapes=[
            pltpu.VMEM((_G, _H), jnp.float32),
            pltpu.VMEM((_G, 1), jnp.float32),
        ],
    )(h, batch2d, wlin, blin)


def kernel(x, edge_index, batch, W1r, W1s, b1, W2r, W2s, b2, W3r, W3s, b3, Wlin, blin):
    src4 = edge_index[0].reshape(_NW, 2, _HCH, _EBATCH).transpose(1, 0, 2, 3)
    dst4 = edge_index[1].reshape(_NW, 2, _HCH, _EBATCH).transpose(1, 0, 2, 3)
    sd = jnp.concatenate([src4, dst4], axis=2)
    zeros = jnp.zeros((_ZR, _H), jnp.float32)
    batch2d = batch.reshape(_N, 1).astype(jnp.int32)

    h = x
    for wr, ws, b, relu in (
        (W1r, W1s, b1, True),
        (W2r, W2s, b2, True),
        (W3r, W3s, b3, False),
    ):
        accs = _seg_sum(h, sd, zeros)
        h = _fused_linear(accs[0], accs[1], h, wr, ws, b.reshape(1, _H), relu)
    return _pool(h, batch2d, Wlin, blin.reshape(1, _C))


# speedup vs baseline: 10.8829x; 1.0235x over previous
import functools

import jax
import jax.numpy as jnp
from jax import lax
from jax.experimental import pallas as pl
from jax.experimental.pallas import tpu as pltpu
from jax.experimental.pallas import tpu_sc as plsc

_N = 10000
_E = 320000
_H = 128
_G = 64
_C = 10

_NC = 2
_NS = 16
_NW = _NC * _NS
_EPT = _E // _NW
_EBATCH = 125
_NCH = _EPT // _EBATCH
_HCH = _NCH // 2
_ZR = _N // _NS

_mesh = plsc.VectorSubcoreMesh(
    core_axis_name="c", subcore_axis_name="s", num_cores=_NC, num_subcores=_NS
)


@functools.partial(
    pl.kernel,
    out_type=jax.ShapeDtypeStruct((_NC, _N, _H), jnp.float32),
    mesh=_mesh,
    scratch_types=[
        pltpu.VMEM((2 * _HCH, _EBATCH), jnp.int32),
        pltpu.VMEM((2, _EBATCH, _H), jnp.float32),
        pltpu.VMEM_SHARED((_N, _H), jnp.float32),
        pltpu.SemaphoreType.DMA((2,)),
    ],
)
def _seg_sum(h, sd, zeros, out, idx_v, rows_v, acc, sem):
    c = lax.axis_index("c")
    s = lax.axis_index("s")
    wid = c * _NS + s
    pltpu.sync_copy(zeros, acc.at[pl.ds(s * _ZR, _ZR)])
    plsc.subcore_barrier()

    for half in range(2):
        pltpu.sync_copy(sd.at[half].at[wid], idx_v)
        pltpu.async_copy(h.at[idx_v.at[0]], rows_v.at[0], sem.at[0])

        def body(j, carry):
            nxt = j + 1

            @pl.when(nxt < _HCH)
            def _():
                nb = lax.rem(nxt, 2)
                pltpu.async_copy(h.at[idx_v.at[nxt]], rows_v.at[nb], sem.at[nb])

            jb = lax.rem(j, 2)
            pltpu.make_async_copy(
                h.at[idx_v.at[j]], rows_v.at[jb], sem.at[jb]
            ).wait()
            pltpu.sync_copy(rows_v.at[jb], acc.at[idx_v.at[_HCH + j]], add=True)
            return carry

        lax.fori_loop(0, _HCH, body, 0)
    plsc.subcore_barrier()

    @pl.when(s == 0)
    def _():
        pltpu.sync_copy(acc, out.at[c])


_BR = 1000


def _lin_body(a0, a1, x, wr, ws, b, o, *, relu):
    y = jnp.dot(a0[...] + a1[...], wr[...], preferred_element_type=jnp.float32)
    y = y + jnp.dot(x[...], ws[...], preferred_element_type=jnp.float32)
    y = y + b[...]
    if relu:
        y = jnp.maximum(y, 0.0)
    o[...] = y


def _fused_linear(a0, a1, x, wr, ws, b, relu):
    return pl.pallas_call(
        functools.partial(_lin_body, relu=relu),
        grid=(_N // _BR,),
        in_specs=[
            pl.BlockSpec((_BR, _H), lambda i: (i, 0)),
            pl.BlockSpec((_BR, _H), lambda i: (i, 0)),
            pl.BlockSpec((_BR, _H), lambda i: (i, 0)),
            pl.BlockSpec((_H, _H), lambda i: (0, 0)),
            pl.BlockSpec((_H, _H), lambda i: (0, 0)),
            pl.BlockSpec((1, _H), lambda i: (0, 0)),
        ],
        out_specs=pl.BlockSpec((_BR, _H), lambda i: (i, 0)),
        out_shape=jax.ShapeDtypeStruct((_N, _H), jnp.float32),
    )(a0, a1, x, wr, ws, b)


def _last_body(a0, a1, x, wr, ws, b, batch, wlin, blin, o, sums, counts):
    i = pl.program_id(0)

    @pl.when(i == 0)
    def _():
        sums[...] = jnp.zeros_like(sums)
        counts[...] = jnp.zeros_like(counts)

    y = jnp.dot(a0[...] + a1[...], wr[...], preferred_element_type=jnp.float32)
    y = y + jnp.dot(x[...], ws[...], preferred_element_type=jnp.float32)
    y = y + b[...]
    oh = (lax.broadcasted_iota(jnp.int32, (_BR, _G), 1) == batch[...]).astype(
        jnp.float32
    )
    sums[...] += lax.dot_general(
        oh, y, (((0,), (0,)), ((), ())), preferred_element_type=jnp.float32
    )
    counts[...] += lax.dot_general(
        oh,
        jnp.ones((_BR, 1), jnp.float32),
        (((0,), (0,)), ((), ())),
        preferred_element_type=jnp.float32,
    )

    @pl.when(i == pl.num_programs(0) - 1)
    def _():
        pooled = sums[...] / jnp.maximum(counts[...], 1.0)
        o[...] = (
            jnp.dot(pooled, wlin[...], preferred_element_type=jnp.float32) + blin[...]
        )


def _last_layer_pool(a0, a1, x, wr, ws, b, batch2d, wlin, blin):
    return pl.pallas_call(
        _last_body,
        grid=(_N // _BR,),
        in_specs=[
            pl.BlockSpec((_BR, _H), lambda i: (i, 0)),
            pl.BlockSpec((_BR, _H), lambda i: (i, 0)),
            pl.BlockSpec((_BR, _H), lambda i: (i, 0)),
            pl.BlockSpec((_H, _H), lambda i: (0, 0)),
            pl.BlockSpec((_H, _H), lambda i: (0, 0)),
            pl.BlockSpec((1, _H), lambda i: (0, 0)),
            pl.BlockSpec((_BR, 1), lambda i: (i, 0)),
            pl.BlockSpec((_H, _C), lambda i: (0, 0)),
            pl.BlockSpec((1, _C), lambda i: (0, 0)),
        ],
        out_specs=pl.BlockSpec((_G, _C), lambda i: (0, 0)),
        out_shape=jax.ShapeDtypeStruct((_G, _C), jnp.float32),
        scratch_shapes=[
            pltpu.VMEM((_G, _H), jnp.float32),
            pltpu.VMEM((_G, 1), jnp.float32),
        ],
    )(a0, a1, x, wr, ws, b, batch2d, wlin, blin)


def kernel(x, edge_index, batch, W1r, W1s, b1, W2r, W2s, b2, W3r, W3s, b3, Wlin, blin):
    src4 = edge_index[0].reshape(_NW, 2, _HCH, _EBATCH).transpose(1, 0, 2, 3)
    dst4 = edge_index[1].reshape(_NW, 2, _HCH, _EBATCH).transpose(1, 0, 2, 3)
    sd = jnp.concatenate([src4, dst4], axis=2)
    zeros = jnp.zeros((_ZR, _H), jnp.float32)
    batch2d = batch.reshape(_N, 1).astype(jnp.int32)

    h = x
    for wr, ws, b in ((W1r, W1s, b1), (W2r, W2s, b2)):
        accs = _seg_sum(h, sd, zeros)
        h = _fused_linear(accs[0], accs[1], h, wr, ws, b.reshape(1, _H), True)
    accs = _seg_sum(h, sd, zeros)
    return _last_layer_pool(
        accs[0], accs[1], h, W3r, W3s, b3.reshape(1, _H),
        batch2d, Wlin, blin.reshape(1, _C),
    )


# speedup vs baseline: 10.9014x; 1.0017x over previous
import functools

import jax
import jax.numpy as jnp
from jax import lax
from jax.experimental import pallas as pl
from jax.experimental.pallas import tpu as pltpu
from jax.experimental.pallas import tpu_sc as plsc

_N = 10000
_E = 320000
_H = 128
_G = 64
_C = 10

_NC = 2
_NS = 16
_NW = _NC * _NS
_EPT = _E // _NW
_EBATCH = 125
_NCH = _EPT // _EBATCH
_HCH = _NCH // 2
_ZR = _N // _NS
_FR = 632
_FL = _N - (_NS - 1) * _FR

_mesh = plsc.VectorSubcoreMesh(
    core_axis_name="c", subcore_axis_name="s", num_cores=_NC, num_subcores=_NS
)


@functools.partial(
    pl.kernel,
    out_type=jax.ShapeDtypeStruct((_NC, _N, _H), jnp.float32),
    mesh=_mesh,
    scratch_types=[
        pltpu.VMEM((2 * _HCH, _EBATCH), jnp.int32),
        pltpu.VMEM((2, _EBATCH, _H), jnp.float32),
        pltpu.VMEM_SHARED((_N, _H), jnp.float32),
        pltpu.SemaphoreType.DMA((2,)),
    ],
)
def _seg_sum(h, sd, zeros, out, idx_v, rows_v, acc, sem):
    c = lax.axis_index("c")
    s = lax.axis_index("s")
    wid = c * _NS + s
    pltpu.sync_copy(zeros, acc.at[pl.ds(s * _ZR, _ZR)])
    plsc.subcore_barrier()

    for half in range(2):
        pltpu.sync_copy(sd.at[half].at[wid], idx_v)
        pltpu.async_copy(h.at[idx_v.at[0]], rows_v.at[0], sem.at[0])

        def body(j, carry):
            nxt = j + 1

            @pl.when(nxt < _HCH)
            def _():
                nb = lax.rem(nxt, 2)
                pltpu.async_copy(h.at[idx_v.at[nxt]], rows_v.at[nb], sem.at[nb])

            jb = lax.rem(j, 2)
            pltpu.make_async_copy(
                h.at[idx_v.at[j]], rows_v.at[jb], sem.at[jb]
            ).wait()
            pltpu.sync_copy(rows_v.at[jb], acc.at[idx_v.at[_HCH + j]], add=True)
            return carry

        lax.fori_loop(0, _HCH, body, 0)
    plsc.subcore_barrier()

    @pl.when(s < _NS - 1)
    def _():
        pltpu.sync_copy(
            acc.at[pl.ds(s * _FR, _FR)], out.at[c].at[pl.ds(s * _FR, _FR)]
        )

    @pl.when(s == _NS - 1)
    def _():
        pltpu.sync_copy(
            acc.at[pl.ds((_NS - 1) * _FR, _FL)],
            out.at[c].at[pl.ds((_NS - 1) * _FR, _FL)],
        )


_BR = 1000


def _lin_body(a0, a1, x, wr, ws, b, o, *, relu):
    y = jnp.dot(a0[...] + a1[...], wr[...], preferred_element_type=jnp.float32)
    y = y + jnp.dot(x[...], ws[...], preferred_element_type=jnp.float32)
    y = y + b[...]
    if relu:
        y = jnp.maximum(y, 0.0)
    o[...] = y


def _fused_linear(a0, a1, x, wr, ws, b, relu):
    return pl.pallas_call(
        functools.partial(_lin_body, relu=relu),
        grid=(_N // _BR,),
        in_specs=[
            pl.BlockSpec((_BR, _H), lambda i: (i, 0)),
            pl.BlockSpec((_BR, _H), lambda i: (i, 0)),
            pl.BlockSpec((_BR, _H), lambda i: (i, 0)),
            pl.BlockSpec((_H, _H), lambda i: (0, 0)),
            pl.BlockSpec((_H, _H), lambda i: (0, 0)),
            pl.BlockSpec((1, _H), lambda i: (0, 0)),
        ],
        out_specs=pl.BlockSpec((_BR, _H), lambda i: (i, 0)),
        out_shape=jax.ShapeDtypeStruct((_N, _H), jnp.float32),
    )(a0, a1, x, wr, ws, b)


def _last_body(a0, a1, x, wr, ws, b, batch, wlin, blin, o, sums, counts):
    i = pl.program_id(0)

    @pl.when(i == 0)
    def _():
        sums[...] = jnp.zeros_like(sums)
        counts[...] = jnp.zeros_like(counts)

    y = jnp.dot(a0[...] + a1[...], wr[...], preferred_element_type=jnp.float32)
    y = y + jnp.dot(x[...], ws[...], preferred_element_type=jnp.float32)
    y = y + b[...]
    oh = (lax.broadcasted_iota(jnp.int32, (_BR, _G), 1) == batch[...]).astype(
        jnp.float32
    )
    sums[...] += lax.dot_general(
        oh, y, (((0,), (0,)), ((), ())), preferred_element_type=jnp.float32
    )
    counts[...] += lax.dot_general(
        oh,
        jnp.ones((_BR, 1), jnp.float32),
        (((0,), (0,)), ((), ())),
        preferred_element_type=jnp.float32,
    )

    @pl.when(i == pl.num_programs(0) - 1)
    def _():
        pooled = sums[...] / jnp.maximum(counts[...], 1.0)
        o[...] = (
            jnp.dot(pooled, wlin[...], preferred_element_type=jnp.float32) + blin[...]
        )


def _last_layer_pool(a0, a1, x, wr, ws, b, batch2d, wlin, blin):
    return pl.pallas_call(
        _last_body,
        grid=(_N // _BR,),
        in_specs=[
            pl.BlockSpec((_BR, _H), lambda i: (i, 0)),
            pl.BlockSpec((_BR, _H), lambda i: (i, 0)),
            pl.BlockSpec((_BR, _H), lambda i: (i, 0)),
            pl.BlockSpec((_H, _H), lambda i: (0, 0)),
            pl.BlockSpec((_H, _H), lambda i: (0, 0)),
            pl.BlockSpec((1, _H), lambda i: (0, 0)),
            pl.BlockSpec((_BR, 1), lambda i: (i, 0)),
            pl.BlockSpec((_H, _C), lambda i: (0, 0)),
            pl.BlockSpec((1, _C), lambda i: (0, 0)),
        ],
        out_specs=pl.BlockSpec((_G, _C), lambda i: (0, 0)),
        out_shape=jax.ShapeDtypeStruct((_G, _C), jnp.float32),
        scratch_shapes=[
            pltpu.VMEM((_G, _H), jnp.float32),
            pltpu.VMEM((_G, 1), jnp.float32),
        ],
    )(a0, a1, x, wr, ws, b, batch2d, wlin, blin)


def kernel(x, edge_index, batch, W1r, W1s, b1, W2r, W2s, b2, W3r, W3s, b3, Wlin, blin):
    src4 = edge_index[0].reshape(_NW, 2, _HCH, _EBATCH).transpose(1, 0, 2, 3)
    dst4 = edge_index[1].reshape(_NW, 2, _HCH, _EBATCH).transpose(1, 0, 2, 3)
    sd = jnp.concatenate([src4, dst4], axis=2)
    zeros = jnp.zeros((_ZR, _H), jnp.float32)
    batch2d = batch.reshape(_N, 1).astype(jnp.int32)

    h = x
    for wr, ws, b in ((W1r, W1s, b1), (W2r, W2s, b2)):
        accs = _seg_sum(h, sd, zeros)
        h = _fused_linear(accs[0], accs[1], h, wr, ws, b.reshape(1, _H), True)
    accs = _seg_sum(h, sd, zeros)
    return _last_layer_pool(
        accs[0], accs[1], h, W3r, W3s, b3.reshape(1, _H),
        batch2d, Wlin, blin.reshape(1, _C),
    )
